# baseline (speedup 1.0000x reference)
import functools

import jax
import jax.numpy as jnp
from jax import lax
from jax.experimental import pallas as pl
from jax.experimental.pallas import tpu as pltpu
from jax.experimental.pallas import tpu_sc as plsc

N = 10000
NP = 10240
E = 160000
EP = 163840
IN_DIM = 1280
H = 256
HH = 128
G = 8
BR = 1024
NBLK = NP // BR
NC, NS, L = 2, 16, 16
EPT = EP // NS
CH = 128
RPT = NP // NS
DEG_CH = 1024

_MESH = plsc.VectorSubcoreMesh(
    core_axis_name="c", subcore_axis_name="s", num_cores=NC, num_subcores=NS)
_SC_PARAMS = pltpu.CompilerParams(needs_layout_passes=False)


@functools.partial(
    pl.kernel,
    out_type=jax.ShapeDtypeStruct((NC * NS, NP), jnp.float32),
    mesh=_MESH,
    compiler_params=_SC_PARAMS,
    scratch_types=[
        pltpu.VMEM((NP,), jnp.float32),
        pltpu.VMEM((DEG_CH,), jnp.int32),
        pltpu.VMEM((DEG_CH,), jnp.float32),
    ],
)
def _deg_kernel(col_hbm, ew_hbm, deg_hbm, acc_v, col_v, ew_v):
    wid = lax.axis_index("s") * NC + lax.axis_index("c")
    ept = EP // (NC * NS)
    zero = jnp.zeros((L,), jnp.float32)

    @pl.loop(0, NP, step=L)
    def _zero(o):
        acc_v[pl.ds(o, L)] = zero

    @pl.loop(0, ept, step=DEG_CH)
    def _chunk(off):
        o = wid * ept + off
        pltpu.sync_copy(col_hbm.at[pl.ds(o, DEG_CH)], col_v)
        pltpu.sync_copy(ew_hbm.at[pl.ds(o, DEG_CH)], ew_v)
        for g in range(DEG_CH // L):
            cv = col_v[pl.ds(g * L, L)]
            wv = ew_v[pl.ds(g * L, L)]
            plsc.addupdate_scatter(acc_v, [cv], wv)

    pltpu.sync_copy(acc_v, deg_hbm.at[wid])


NCH = EP // CH
CAPH = 6144
NCHH = CAPH // CH


@functools.partial(
    pl.kernel,
    out_type=(jax.ShapeDtypeStruct((NS, NC, CAPH), jnp.int32),
              jax.ShapeDtypeStruct((NS, NC, CAPH), jnp.float32)),
    mesh=_MESH,
    compiler_params=_SC_PARAMS,
    scratch_types=[
        pltpu.VMEM((CAPH,), jnp.int32),
        pltpu.VMEM((CAPH,), jnp.float32),
        pltpu.VMEM((3, CH), jnp.int32),
        pltpu.VMEM((3, CH), jnp.int32),
        pltpu.SemaphoreType.DMA,
        pltpu.SemaphoreType.DMA,
    ],
)
def _prep_kernel(edata_hbm, bpk_hbm, bew_hbm,
                 pk_buf, ew_buf, ed0, ed1, se0, se1):
    c = lax.axis_index("c")
    s = lax.axis_index("s")
    nch = NCH // NC
    base = c * nch
    lo = s * RPT
    eds = (ed0, ed1)
    sems = (se0, se1)

    zi = jnp.zeros((L,), jnp.int32)
    zf = jnp.zeros((L,), jnp.float32)

    @pl.loop(0, CAPH, step=L)
    def _zero(o):
        pk_buf[pl.ds(o, L)] = zi
        ew_buf[pl.ds(o, L)] = zf

    pltpu.async_copy(edata_hbm.at[base], ed0, se0)
    pltpu.async_copy(edata_hbm.at[base + 1], ed1, se1)

    @pl.loop(0, nch, step=2, init_carry=0)
    def _scan(g0, ptr):
        for b in range(2):
            g = g0 + b
            pltpu.make_async_copy(edata_hbm.at[0], eds[b], sems[b]).wait()
            ed = eds[b]
            for gb in range(CH // L):
                rowv = ed[0, pl.ds(gb * L, L)]
                colv = ed[1, pl.ds(gb * L, L)]
                ewv = plsc.bitcast(ed[2, pl.ds(gb * L, L)], jnp.float32)
                m = jnp.logical_and(colv >= lo, colv < lo + RPT)
                pk = rowv | ((colv - lo) << 14)
                p = jnp.minimum(ptr, CAPH - L)
                plsc.store_compressed(pk_buf.at[pl.ds(p, L)], pk, mask=m)
                plsc.store_compressed(ew_buf.at[pl.ds(p, L)], ewv, mask=m)
                cnt = plsc.all_reduce_population_count(m)
                ptr = ptr + cnt[0]

            @pl.when(g + 2 < nch)
            def _():
                pltpu.async_copy(edata_hbm.at[base + g + 2], eds[b], sems[b])

        return ptr

    pltpu.sync_copy(pk_buf, bpk_hbm.at[s, c])
    pltpu.sync_copy(ew_buf, bew_hbm.at[s, c])


@functools.partial(
    pl.kernel,
    out_type=jax.ShapeDtypeStruct((NC * NP * HH,), jnp.float32),
    mesh=_MESH,
    compiler_params=_SC_PARAMS,
    scratch_types=[
        pltpu.VMEM((RPT * HH,), jnp.float32),
        pltpu.VMEM((CAPH,), jnp.int32),
        pltpu.VMEM((CAPH,), jnp.float32),
        pltpu.VMEM((CH,), jnp.int32),
        pltpu.VMEM((CH,), jnp.int32),
        pltpu.VMEM((CH, HH), jnp.float32),
        pltpu.VMEM((CH, HH), jnp.float32),
        pltpu.SemaphoreType.DMA,
        pltpu.SemaphoreType.DMA,
    ],
)
def _agg_kernel(y_hbm, bpk_hbm, bew_hbm, out_hbm,
                acc_v, pk_v, ew_v, gi0, gi1, rows0, rows1, sg0, sg1):
    c = lax.axis_index("c")
    s = lax.axis_index("s")
    coff = c * NP
    gis = (gi0, gi1)
    rows = (rows0, rows1)
    sgs = (sg0, sg1)

    zero = jnp.zeros((L,), jnp.float32)

    @pl.loop(0, RPT * HH, step=L)
    def _zero(o):
        acc_v[pl.ds(o, L)] = zero

    for h in range(NC):
        pltpu.sync_copy(bpk_hbm.at[s, h], pk_v)
        pltpu.sync_copy(bew_hbm.at[s, h], ew_v)

        def _start_gather(g, b):
            gi = gis[b]
            off = g * CH

            @pl.loop(0, CH, step=L)
            def _gx(gb):
                gi[pl.ds(gb, L)] = (pk_v[pl.ds(off + gb, L)] & 16383) + coff

            pltpu.async_copy(y_hbm.at[gi], rows[b], sgs[b])

        _start_gather(0, 0)

        @pl.loop(0, NCHH, step=2)
        def _chunk(g0):
            for b in range(2):
                g = g0 + b

                @pl.when(g + 1 < NCHH)
                def _():
                    _start_gather(g + 1, 1 - b)

                pltpu.make_async_copy(y_hbm.at[gis[b]], rows[b], sgs[b]).wait()
                rv = rows[b]
                off = g * CH

                @plsc.parallel_loop(0, CH, step=L, unroll=4)
                def _acc(gb):
                    pkv = pk_v[pl.ds(off + gb, L)]
                    ew_vec = ew_v[pl.ds(off + gb, L)]
                    cb_vec = (pkv >> 14) << 7
                    for l in range(L):
                        ew_b = jnp.full((L,), ew_vec[l], jnp.float32)
                        cb = cb_vec[l]
                        rrow = rv.at[gb + l]
                        for j16 in range(HH // L):
                            vals = rrow[pl.ds(j16 * L, L)] * ew_b
                            plsc.addupdate(
                                acc_v.at[pl.ds(cb + j16 * L, L)], vals)

    pltpu.sync_copy(acc_v, out_hbm.at[pl.ds((coff + s * RPT) * HH, RPT * HH)])


def _proj_body(x_ref, w_ref, b_ref, deg_ref, h_ref, dis_ref):
    h_ref[...] = (
        jnp.dot(x_ref[...], w_ref[...], preferred_element_type=jnp.float32)
        + b_ref[...])
    deg = jnp.sum(deg_ref[...], axis=0)[:, None]
    dis_ref[...] = lax.rsqrt(deg + 1.0)


def _proj(x, w_in, b_in, deg):
    return pl.pallas_call(
        _proj_body,
        grid=(NBLK,),
        in_specs=[
            pl.BlockSpec((BR, IN_DIM), lambda i: (i, 0)),
            pl.BlockSpec((IN_DIM, H), lambda i: (0, 0)),
            pl.BlockSpec((1, H), lambda i: (0, 0)),
            pl.BlockSpec((NC * NS, BR), lambda i: (0, i)),
        ],
        out_specs=[
            pl.BlockSpec((BR, H), lambda i: (i, 0)),
            pl.BlockSpec((BR, 1), lambda i: (i, 0)),
        ],
        out_shape=[
            jax.ShapeDtypeStruct((NP, H), jnp.float32),
            jax.ShapeDtypeStruct((NP, 1), jnp.float32),
        ],
    )(x, w_in, b_in.reshape(1, H), deg)


def _scaled_mm_body(h_ref, w_ref, dis_ref, y_ref):
    y_ref[...] = dis_ref[...] * jnp.dot(
        h_ref[...], w_ref[...], preferred_element_type=jnp.float32)


def _scaled_mm(h, w, dis):
    return pl.pallas_call(
        _scaled_mm_body,
        grid=(NC, NBLK),
        in_specs=[
            pl.BlockSpec((BR, H), lambda c, i: (i, 0)),
            pl.BlockSpec((H, HH), lambda c, i: (0, c)),
            pl.BlockSpec((BR, 1), lambda c, i: (i, 0)),
        ],
        out_specs=pl.BlockSpec((BR, HH), lambda c, i: (c * NBLK + i, 0)),
        out_shape=jax.ShapeDtypeStruct((NC * NP, HH), jnp.float32),
    )(h, w, dis)


def _layer_mm_body(a0_ref, a1_ref, y0_ref, y1_ref, dis_ref, b_ref, w_ref,
                   out_ref):
    dis = dis_ref[...]
    h0 = jnp.maximum(dis * (a0_ref[...] + y0_ref[...]) + b_ref[0:1, :], 0.0)
    h1 = jnp.maximum(dis * (a1_ref[...] + y1_ref[...]) + b_ref[1:2, :], 0.0)
    acc = jnp.dot(h0, w_ref[0:HH, :], preferred_element_type=jnp.float32)
    acc += jnp.dot(h1, w_ref[HH:H, :], preferred_element_type=jnp.float32)
    out_ref[...] = dis * acc


def _layer_mm(agg, y, dis, b, w):
    return pl.pallas_call(
        _layer_mm_body,
        grid=(NC, NBLK),
        in_specs=[
            pl.BlockSpec((BR, HH), lambda c, i: (i, 0)),
            pl.BlockSpec((BR, HH), lambda c, i: (NBLK + i, 0)),
            pl.BlockSpec((BR, HH), lambda c, i: (i, 0)),
            pl.BlockSpec((BR, HH), lambda c, i: (NBLK + i, 0)),
            pl.BlockSpec((BR, 1), lambda c, i: (i, 0)),
            pl.BlockSpec((2, HH), lambda c, i: (0, 0)),
            pl.BlockSpec((H, HH), lambda c, i: (0, c)),
        ],
        out_specs=pl.BlockSpec((BR, HH), lambda c, i: (c * NBLK + i, 0)),
        out_shape=jax.ShapeDtypeStruct((NC * NP, HH), jnp.float32),
    )(agg, agg, y, y, dis, b.reshape(2, HH), w)


def _pool_body(a_ref, y_ref, dis_ref, b_ref, batch_ref, out_ref, acc, cnt):
    i = pl.program_id(1)
    h3 = jnp.maximum(
        dis_ref[...] * (a_ref[...] + y_ref[...]) + b_ref[...], 0.0)
    onehot = (batch_ref[...] == jnp.arange(G, dtype=jnp.int32)[None, :]
              ).astype(jnp.float32)
    part = lax.dot_general(onehot, h3, (((0,), (0,)), ((), ())),
                           preferred_element_type=jnp.float32)
    cpart = jnp.sum(onehot, axis=0).reshape(G, 1)

    @pl.when(i == 0)
    def _():
        acc[...] = part
        cnt[...] = cpart

    @pl.when(i > 0)
    def _():
        acc[...] += part
        cnt[...] += cpart

    @pl.when(i == NBLK - 1)
    def _():
        out_ref[...] = acc[...] / jnp.clip(cnt[...], 1.0, None)


def _pool(agg, y, dis, b, batch):
    return pl.pallas_call(
        _pool_body,
        grid=(NC, NBLK),
        in_specs=[
            pl.BlockSpec((BR, HH), lambda c, i: (c * NBLK + i, 0)),
            pl.BlockSpec((BR, HH), lambda c, i: (c * NBLK + i, 0)),
            pl.BlockSpec((BR, 1), lambda c, i: (i, 0)),
            pl.BlockSpec((1, HH), lambda c, i: (0, c)),
            pl.BlockSpec((BR, 1), lambda c, i: (i, 0)),
        ],
        out_specs=pl.BlockSpec((G, HH), lambda c, i: (0, c)),
        out_shape=jax.ShapeDtypeStruct((G, H), jnp.float32),
        scratch_shapes=[
            pltpu.VMEM((G, HH), jnp.float32),
            pltpu.VMEM((G, 1), jnp.float32),
        ],
    )(agg, y, dis, b.reshape(1, H), batch)


def kernel(x, edge_index, edge_weight, batch, W_in, b_in, W1, b1, W2, b2):
    row = edge_index[0].astype(jnp.int32)
    col = edge_index[1].astype(jnp.int32)
    pad = EP - E
    row_p = jnp.concatenate([row, jnp.zeros((pad,), jnp.int32)])
    col_p = jnp.concatenate([col, jnp.full((pad,), N, jnp.int32)])
    ew_p = jnp.concatenate([edge_weight.astype(jnp.float32),
                            jnp.zeros((pad,), jnp.float32)])
    x_p = jnp.pad(x, ((0, NP - N), (0, 0)))
    batch_p = jnp.concatenate(
        [batch.astype(jnp.int32), jnp.full((NP - N,), G, jnp.int32)]
    ).reshape(NP, 1)

    nch_tot = EP // CH
    ew_i = lax.bitcast_convert_type(ew_p, jnp.int32)
    edata = jnp.stack([row_p.reshape(nch_tot, CH), col_p.reshape(nch_tot, CH),
                       ew_i.reshape(nch_tot, CH)], axis=1)

    deg = _deg_kernel(col_p, ew_p)
    bpk, bew = _prep_kernel(edata)
    h1, dis = _proj(x_p, W_in, b_in, deg)
    y1 = _scaled_mm(h1, W1, dis)
    agg1 = _agg_kernel(y1, bpk, bew).reshape(NC * NP, HH)
    y2 = _layer_mm(agg1, y1, dis, b1, W2)
    agg2 = _agg_kernel(y2, bpk, bew).reshape(NC * NP, HH)
    return _pool(agg2, y2, dis, b2, batch_p)


# speedup vs baseline: 1.0725x; 1.0725x over previous
import functools

import jax
import jax.numpy as jnp
from jax import lax
from jax.experimental import pallas as pl
from jax.experimental.pallas import tpu as pltpu
from jax.experimental.pallas import tpu_sc as plsc

N = 10000
NP = 10240
E = 160000
EP = 163840
IN_DIM = 1280
H = 256
HH = 128
G = 8
BR = 1024
NBLK = NP // BR
NC, NS, L = 2, 16, 16
EPT = EP // NS
CH = 128
RPT = NP // NS
DEG_CH = 1024

_MESH = plsc.VectorSubcoreMesh(
    core_axis_name="c", subcore_axis_name="s", num_cores=NC, num_subcores=NS)
_SC_PARAMS = pltpu.CompilerParams(needs_layout_passes=False)


@functools.partial(
    pl.kernel,
    out_type=jax.ShapeDtypeStruct((NC * NS, NP), jnp.float32),
    mesh=_MESH,
    compiler_params=_SC_PARAMS,
    scratch_types=[
        pltpu.VMEM((NP,), jnp.float32),
        pltpu.VMEM((DEG_CH,), jnp.int32),
        pltpu.VMEM((DEG_CH,), jnp.float32),
    ],
)
def _deg_kernel(col_hbm, ew_hbm, deg_hbm, acc_v, col_v, ew_v):
    wid = lax.axis_index("s") * NC + lax.axis_index("c")
    ept = EP // (NC * NS)
    zero = jnp.zeros((L,), jnp.float32)

    @pl.loop(0, NP, step=L)
    def _zero(o):
        acc_v[pl.ds(o, L)] = zero

    @pl.loop(0, ept, step=DEG_CH)
    def _chunk(off):
        o = wid * ept + off
        pltpu.sync_copy(col_hbm.at[pl.ds(o, DEG_CH)], col_v)
        pltpu.sync_copy(ew_hbm.at[pl.ds(o, DEG_CH)], ew_v)
        for g in range(DEG_CH // L):
            cv = col_v[pl.ds(g * L, L)]
            wv = ew_v[pl.ds(g * L, L)]
            plsc.addupdate_scatter(acc_v, [cv], wv)

    pltpu.sync_copy(acc_v, deg_hbm.at[wid])


NCH = EP // CH
CAPH = 6144
NCHH = CAPH // CH


@functools.partial(
    pl.kernel,
    out_type=(jax.ShapeDtypeStruct((NS, NC, CAPH), jnp.int32),
              jax.ShapeDtypeStruct((NS, NC, CAPH), jnp.float32)),
    mesh=_MESH,
    compiler_params=_SC_PARAMS,
    scratch_types=[
        pltpu.VMEM((CAPH,), jnp.int32),
        pltpu.VMEM((CAPH,), jnp.float32),
        pltpu.VMEM((3, CH), jnp.int32),
        pltpu.VMEM((3, CH), jnp.int32),
        pltpu.SemaphoreType.DMA,
        pltpu.SemaphoreType.DMA,
    ],
)
def _prep_kernel(edata_hbm, bpk_hbm, bew_hbm,
                 pk_buf, ew_buf, ed0, ed1, se0, se1):
    c = lax.axis_index("c")
    s = lax.axis_index("s")
    nch = NCH // NC
    base = c * nch
    lo = s * RPT
    eds = (ed0, ed1)
    sems = (se0, se1)

    zi = jnp.zeros((L,), jnp.int32)
    zf = jnp.zeros((L,), jnp.float32)

    @pl.loop(0, CAPH, step=L)
    def _zero(o):
        pk_buf[pl.ds(o, L)] = zi
        ew_buf[pl.ds(o, L)] = zf

    pltpu.async_copy(edata_hbm.at[base], ed0, se0)
    pltpu.async_copy(edata_hbm.at[base + 1], ed1, se1)

    @pl.loop(0, nch, step=2, init_carry=0)
    def _scan(g0, ptr):
        for b in range(2):
            g = g0 + b
            pltpu.make_async_copy(edata_hbm.at[0], eds[b], sems[b]).wait()
            ed = eds[b]
            for gb in range(CH // L):
                rowv = ed[0, pl.ds(gb * L, L)]
                colv = ed[1, pl.ds(gb * L, L)]
                ewv = plsc.bitcast(ed[2, pl.ds(gb * L, L)], jnp.float32)
                m = jnp.logical_and(colv >= lo, colv < lo + RPT)
                pk = rowv | ((colv - lo) << 14)
                p = jnp.minimum(ptr, CAPH - L)
                plsc.store_compressed(pk_buf.at[pl.ds(p, L)], pk, mask=m)
                plsc.store_compressed(ew_buf.at[pl.ds(p, L)], ewv, mask=m)
                cnt = plsc.all_reduce_population_count(m)
                ptr = ptr + cnt[0]

            @pl.when(g + 2 < nch)
            def _():
                pltpu.async_copy(edata_hbm.at[base + g + 2], eds[b], sems[b])

        return ptr

    pltpu.sync_copy(pk_buf, bpk_hbm.at[s, c])
    pltpu.sync_copy(ew_buf, bew_hbm.at[s, c])


@functools.partial(
    pl.kernel,
    out_type=jax.ShapeDtypeStruct((NC * NP * HH,), jnp.float32),
    mesh=_MESH,
    compiler_params=_SC_PARAMS,
    scratch_types=[
        pltpu.VMEM((RPT * HH,), jnp.float32),
        pltpu.VMEM((CAPH,), jnp.int32),
        pltpu.VMEM((CAPH,), jnp.float32),
        pltpu.VMEM((CH,), jnp.int32),
        pltpu.VMEM((CH,), jnp.int32),
        pltpu.VMEM((CH, HH), jnp.float32),
        pltpu.VMEM((CH, HH), jnp.float32),
        pltpu.SemaphoreType.DMA,
        pltpu.SemaphoreType.DMA,
    ],
)
def _agg_kernel(y_hbm, bpk_hbm, bew_hbm, out_hbm,
                acc_v, pk_v, ew_v, gi0, gi1, rows0, rows1, sg0, sg1):
    c = lax.axis_index("c")
    s = lax.axis_index("s")
    coff = c * NP
    gis = (gi0, gi1)
    rows = (rows0, rows1)
    sgs = (sg0, sg1)

    zero = jnp.zeros((L,), jnp.float32)

    @pl.loop(0, RPT * HH, step=L)
    def _zero(o):
        acc_v[pl.ds(o, L)] = zero

    for h in range(NC):
        pltpu.sync_copy(bpk_hbm.at[s, h], pk_v)
        pltpu.sync_copy(bew_hbm.at[s, h], ew_v)

        def _start_gather(g, b):
            gi = gis[b]
            off = g * CH

            @pl.loop(0, CH, step=L)
            def _gx(gb):
                gi[pl.ds(gb, L)] = (pk_v[pl.ds(off + gb, L)] & 16383) + coff

            pltpu.async_copy(y_hbm.at[gi], rows[b], sgs[b])

        _start_gather(0, 0)

        @pl.loop(0, NCHH, step=2)
        def _chunk(g0):
            for b in range(2):
                g = g0 + b

                @pl.when(g + 1 < NCHH)
                def _():
                    _start_gather(g + 1, 1 - b)

                pltpu.make_async_copy(y_hbm.at[gis[b]], rows[b], sgs[b]).wait()
                rv = rows[b]
                off = g * CH

                @plsc.parallel_loop(0, CH, step=L, unroll=2)
                def _acc(gb):
                    pkv = pk_v[pl.ds(off + gb, L)]
                    ew_vec = ew_v[pl.ds(off + gb, L)]
                    cb_vec = (pkv >> 14) << 7
                    for l in range(L):
                        ew_b = jnp.full((L,), ew_vec[l], jnp.float32)
                        cb = cb_vec[l]
                        rrow = rv.at[gb + l]
                        for j16 in range(HH // L):
                            vals = rrow[pl.ds(j16 * L, L)] * ew_b
                            plsc.addupdate(
                                acc_v.at[pl.ds(cb + j16 * L, L)], vals)

    pltpu.sync_copy(acc_v, out_hbm.at[pl.ds((coff + s * RPT) * HH, RPT * HH)])


def _proj_body(x_ref, w_ref, b_ref, deg_ref, h_ref, dis_ref):
    h_ref[...] = (
        jnp.dot(x_ref[...], w_ref[...], preferred_element_type=jnp.float32)
        + b_ref[...])
    deg = jnp.sum(deg_ref[...], axis=0)[:, None]
    dis_ref[...] = lax.rsqrt(deg + 1.0)


def _proj(x, w_in, b_in, deg):
    return pl.pallas_call(
        _proj_body,
        grid=(NBLK,),
        in_specs=[
            pl.BlockSpec((BR, IN_DIM), lambda i: (i, 0)),
            pl.BlockSpec((IN_DIM, H), lambda i: (0, 0)),
            pl.BlockSpec((1, H), lambda i: (0, 0)),
            pl.BlockSpec((NC * NS, BR), lambda i: (0, i)),
        ],
        out_specs=[
            pl.BlockSpec((BR, H), lambda i: (i, 0)),
            pl.BlockSpec((BR, 1), lambda i: (i, 0)),
        ],
        out_shape=[
            jax.ShapeDtypeStruct((NP, H), jnp.float32),
            jax.ShapeDtypeStruct((NP, 1), jnp.float32),
        ],
    )(x, w_in, b_in.reshape(1, H), deg)


def _scaled_mm_body(h_ref, w_ref, dis_ref, y_ref):
    y_ref[...] = dis_ref[...] * jnp.dot(
        h_ref[...], w_ref[...], preferred_element_type=jnp.float32)


def _scaled_mm(h, w, dis):
    return pl.pallas_call(
        _scaled_mm_body,
        grid=(NC, NBLK),
        in_specs=[
            pl.BlockSpec((BR, H), lambda c, i: (i, 0)),
            pl.BlockSpec((H, HH), lambda c, i: (0, c)),
            pl.BlockSpec((BR, 1), lambda c, i: (i, 0)),
        ],
        out_specs=pl.BlockSpec((BR, HH), lambda c, i: (c * NBLK + i, 0)),
        out_shape=jax.ShapeDtypeStruct((NC * NP, HH), jnp.float32),
    )(h, w, dis)


def _layer_mm_body(a0_ref, a1_ref, y0_ref, y1_ref, dis_ref, b_ref, w_ref,
                   out_ref):
    dis = dis_ref[...]
    h0 = jnp.maximum(dis * (a0_ref[...] + y0_ref[...]) + b_ref[0:1, :], 0.0)
    h1 = jnp.maximum(dis * (a1_ref[...] + y1_ref[...]) + b_ref[1:2, :], 0.0)
    acc = jnp.dot(h0, w_ref[0:HH, :], preferred_element_type=jnp.float32)
    acc += jnp.dot(h1, w_ref[HH:H, :], preferred_element_type=jnp.float32)
    out_ref[...] = dis * acc


def _layer_mm(agg, y, dis, b, w):
    return pl.pallas_call(
        _layer_mm_body,
        grid=(NC, NBLK),
        in_specs=[
            pl.BlockSpec((BR, HH), lambda c, i: (i, 0)),
            pl.BlockSpec((BR, HH), lambda c, i: (NBLK + i, 0)),
            pl.BlockSpec((BR, HH), lambda c, i: (i, 0)),
            pl.BlockSpec((BR, HH), lambda c, i: (NBLK + i, 0)),
            pl.BlockSpec((BR, 1), lambda c, i: (i, 0)),
            pl.BlockSpec((2, HH), lambda c, i: (0, 0)),
            pl.BlockSpec((H, HH), lambda c, i: (0, c)),
        ],
        out_specs=pl.BlockSpec((BR, HH), lambda c, i: (c * NBLK + i, 0)),
        out_shape=jax.ShapeDtypeStruct((NC * NP, HH), jnp.float32),
    )(agg, agg, y, y, dis, b.reshape(2, HH), w)


def _pool_body(a_ref, y_ref, dis_ref, b_ref, batch_ref, out_ref, acc, cnt):
    i = pl.program_id(1)
    h3 = jnp.maximum(
        dis_ref[...] * (a_ref[...] + y_ref[...]) + b_ref[...], 0.0)
    onehot = (batch_ref[...] == jnp.arange(G, dtype=jnp.int32)[None, :]
              ).astype(jnp.float32)
    part = lax.dot_general(onehot, h3, (((0,), (0,)), ((), ())),
                           preferred_element_type=jnp.float32)
    cpart = jnp.sum(onehot, axis=0).reshape(G, 1)

    @pl.when(i == 0)
    def _():
        acc[...] = part
        cnt[...] = cpart

    @pl.when(i > 0)
    def _():
        acc[...] += part
        cnt[...] += cpart

    @pl.when(i == NBLK - 1)
    def _():
        out_ref[...] = acc[...] / jnp.clip(cnt[...], 1.0, None)


def _pool(agg, y, dis, b, batch):
    return pl.pallas_call(
        _pool_body,
        grid=(NC, NBLK),
        in_specs=[
            pl.BlockSpec((BR, HH), lambda c, i: (c * NBLK + i, 0)),
            pl.BlockSpec((BR, HH), lambda c, i: (c * NBLK + i, 0)),
            pl.BlockSpec((BR, 1), lambda c, i: (i, 0)),
            pl.BlockSpec((1, HH), lambda c, i: (0, c)),
            pl.BlockSpec((BR, 1), lambda c, i: (i, 0)),
        ],
        out_specs=pl.BlockSpec((G, HH), lambda c, i: (0, c)),
        out_shape=jax.ShapeDtypeStruct((G, H), jnp.float32),
        scratch_shapes=[
            pltpu.VMEM((G, HH), jnp.float32),
            pltpu.VMEM((G, 1), jnp.float32),
        ],
    )(agg, y, dis, b.reshape(1, H), batch)


def kernel(x, edge_index, edge_weight, batch, W_in, b_in, W1, b1, W2, b2):
    row = edge_index[0].astype(jnp.int32)
    col = edge_index[1].astype(jnp.int32)
    pad = EP - E
    row_p = jnp.concatenate([row, jnp.zeros((pad,), jnp.int32)])
    col_p = jnp.concatenate([col, jnp.full((pad,), N, jnp.int32)])
    ew_p = jnp.concatenate([edge_weight.astype(jnp.float32),
                            jnp.zeros((pad,), jnp.float32)])
    x_p = jnp.pad(x, ((0, NP - N), (0, 0)))
    batch_p = jnp.concatenate(
        [batch.astype(jnp.int32), jnp.full((NP - N,), G, jnp.int32)]
    ).reshape(NP, 1)

    nch_tot = EP // CH
    ew_i = lax.bitcast_convert_type(ew_p, jnp.int32)
    edata = jnp.stack([row_p.reshape(nch_tot, CH), col_p.reshape(nch_tot, CH),
                       ew_i.reshape(nch_tot, CH)], axis=1)

    deg = _deg_kernel(col_p, ew_p)
    bpk, bew = _prep_kernel(edata)
    h1, dis = _proj(x_p, W_in, b_in, deg)
    y1 = _scaled_mm(h1, W1, dis)
    agg1 = _agg_kernel(y1, bpk, bew).reshape(NC * NP, HH)
    y2 = _layer_mm(agg1, y1, dis, b1, W2)
    agg2 = _agg_kernel(y2, bpk, bew).reshape(NC * NP, HH)
    return _pool(agg2, y2, dis, b2, batch_p)
